# baseline (speedup 1.0000x reference)
import functools

import jax
import jax.numpy as jnp
from jax import lax
from jax.experimental import pallas as pl
from jax.experimental.pallas import tpu as pltpu
from jax.experimental.pallas import tpu_sc as plsc

N = 10000
E = 160000
NC = 2
NS = 16
RT = 632
NP = NS * RT


def _edge_proj(edge_attr, Ws):
    BE = 2000
    ods = [int(w.shape[1]) for w in Ws]

    def body(ea_ref, *refs):
        w_refs = refs[:len(ods)]
        o_refs = refs[len(ods):]
        a = ea_ref[...]
        for w, o in zip(w_refs, o_refs):
            o[...] = jnp.dot(a, w[...], preferred_element_type=jnp.float32)

    return pl.pallas_call(
        body,
        grid=(E // BE,),
        in_specs=[pl.BlockSpec((BE, 300), lambda i: (i, 0))]
        + [pl.BlockSpec((300, od), lambda i: (0, 0)) for od in ods],
        out_specs=[pl.BlockSpec((BE, od), lambda i: (i, 0)) for od in ods],
        out_shape=[jax.ShapeDtypeStruct((E, od), jnp.float32) for od in ods],
    )(edge_attr, *Ws)


def _node_ab(h, Wi, Wj, be):
    hd = int(h.shape[1])
    od = int(Wi.shape[1])
    BN_ = 2000

    def body(h_ref, wi_ref, wj_ref, be_ref, a_ref, b_ref):
        hh = h_ref[...]
        a_ref[...] = jnp.dot(hh, wi_ref[...], preferred_element_type=jnp.float32) + be_ref[...]
        b_ref[...] = jnp.dot(hh, wj_ref[...], preferred_element_type=jnp.float32)

    return pl.pallas_call(
        body,
        grid=(N // BN_,),
        in_specs=[
            pl.BlockSpec((BN_, hd), lambda i: (i, 0)),
            pl.BlockSpec((hd, od), lambda i: (0, 0)),
            pl.BlockSpec((hd, od), lambda i: (0, 0)),
            pl.BlockSpec((1, od), lambda i: (0, 0)),
        ],
        out_specs=[
            pl.BlockSpec((BN_, od), lambda i: (i, 0)),
            pl.BlockSpec((BN_, od), lambda i: (i, 0)),
        ],
        out_shape=[
            jax.ShapeDtypeStruct((N, od), jnp.float32),
            jax.ShapeDtypeStruct((N, od), jnp.float32),
        ],
    )(h, Wi, Wj, be.reshape(1, od))


def _node_matmul(h, aggm, aggc, xp, Wh, Wa, bn, cs):
    hd = int(h.shape[1])
    od = int(Wa.shape[0])
    nm = int(aggm.shape[0])
    BR = 2000

    def body(h_ref, am_ref, ac_ref, x_ref, wh_ref, wa_ref, bn_ref, cs_ref,
             u_ref, xo_ref):
        aggM = am_ref[0]
        for k in range(1, nm):
            aggM = aggM + am_ref[k]
        aggC = ac_ref[0] + ac_ref[1]
        xo_ref[...] = x_ref[...] + cs_ref[0, 0] * aggC
        z = (jnp.dot(h_ref[...], wh_ref[...], preferred_element_type=jnp.float32)
             + jnp.dot(aggM, wa_ref[...], preferred_element_type=jnp.float32)
             + bn_ref[...])
        u_ref[...] = z * jax.nn.sigmoid(z)

    return pl.pallas_call(
        body,
        grid=(N // BR,),
        in_specs=[
            pl.BlockSpec((BR, hd), lambda i: (i, 0)),
            pl.BlockSpec((nm, BR, od), lambda i: (0, i, 0)),
            pl.BlockSpec((NC, BR, 16), lambda i: (0, i, 0)),
            pl.BlockSpec((BR, 16), lambda i: (i, 0)),
            pl.BlockSpec((hd, od), lambda i: (0, 0)),
            pl.BlockSpec((od, od), lambda i: (0, 0)),
            pl.BlockSpec((1, od), lambda i: (0, 0)),
            pl.BlockSpec((1, 1), lambda i: (0, 0)),
        ],
        out_specs=[
            pl.BlockSpec((BR, od), lambda i: (i, 0)),
            pl.BlockSpec((BR, 16), lambda i: (i, 0)),
        ],
        out_shape=[
            jax.ShapeDtypeStruct((N, od), jnp.float32),
            jax.ShapeDtypeStruct((N, 16), jnp.float32),
        ],
    )(h, aggm, aggc, xp, Wh, Wa, bn.reshape(1, od), cs.reshape(1, 1))


def _bn_act(u, gamma, beta, act):
    od = int(u.shape[1])
    BOD = min(od, 128)

    def body(u_ref, gm_ref, bt_ref, ho_ref):
        uu = u_ref[...]
        mu = jnp.mean(uu, axis=0, keepdims=True)
        d = uu - mu
        var = jnp.mean(d * d, axis=0, keepdims=True)
        v = d * lax.rsqrt(var + 1e-5) * gm_ref[...] + bt_ref[...]
        if act:
            v = v * jax.nn.sigmoid(v)
        ho_ref[...] = v

    return pl.pallas_call(
        body,
        grid=(od // BOD,),
        in_specs=[
            pl.BlockSpec((N, BOD), lambda i: (0, i)),
            pl.BlockSpec((1, BOD), lambda i: (0, i)),
            pl.BlockSpec((1, BOD), lambda i: (0, i)),
        ],
        out_specs=pl.BlockSpec((N, BOD), lambda i: (0, i)),
        out_shape=jax.ShapeDtypeStruct((N, od), jnp.float32),
    )(u, gamma.reshape(1, od), beta.reshape(1, od))


def _zeros16():
    return jnp.zeros((16,), jnp.float32)


def _round_bf16(v):
    b = lax.bitcast_convert_type(v, jnp.int32)
    r = b + 0x7FFF + lax.bitwise_and(lax.shift_right_logical(b, 16), 1)
    r = lax.bitwise_and(r, -65536)
    return lax.bitcast_convert_type(r, jnp.float32)


def _hsum_all(v):
    lanes = lax.iota(jnp.int32, 16)
    for sh in (8, 4, 2, 1):
        idx = lax.bitwise_xor(lanes, sh)
        v = v + v.at[idx].get(mode="promise_in_bounds")
    return v


@functools.cache
def _make_edge_sc(od, C, fused):
    EC = E // NC
    n_chunks = EC // C
    iters = -(-n_chunks // NS)
    K8 = od // 16

    mesh = plsc.VectorSubcoreMesh(core_axis_name="c", subcore_axis_name="s")

    if fused:
        out_type = [jax.ShapeDtypeStruct((NC, NP, od), jnp.float32),
                    jax.ShapeDtypeStruct((NC, NP, 16), jnp.float32)]
    else:
        out_type = [jax.ShapeDtypeStruct((E, od), jnp.float32),
                    jax.ShapeDtypeStruct((NC, NP, 16), jnp.float32)]

    scratch = [
        pltpu.VMEM((C,), jnp.int32),
        pltpu.VMEM((C,), jnp.int32),
        pltpu.VMEM((C, od), jnp.float32),
        pltpu.VMEM((C, od), jnp.float32),
        pltpu.VMEM((C, 16), jnp.float32),
        pltpu.VMEM((C, 16), jnp.float32),
        pltpu.VMEM((C, 16), jnp.float32),
        pltpu.VMEM((2 * od + 16,), jnp.float32),
        pltpu.SemaphoreType.DMA,
    ]
    if fused:
        scratch.append(pltpu.VMEM_SHARED((NP, od), jnp.float32))
    scratch.append(pltpu.VMEM_SHARED((NP, 16), jnp.float32))

    @functools.partial(pl.kernel, out_type=out_type, mesh=mesh,
                       scratch_types=scratch,
                       compiler_params=pltpu.CompilerParams(
                           needs_layout_passes=False,
                           use_tc_tiling_on_sc=False))
    def kern(A, B, Xp, P, src, dst, consts, out1, out2,
             src_v, dst_v, pbuf, mbuf, xibuf, xjbuf, cubuf,
             cv, sem, *shared):
        if fused:
            accM, accC = shared
        else:
            (accC,) = shared
        c = lax.axis_index("c")
        s = lax.axis_index("s")

        pltpu.sync_copy(consts, cv)

        def zrow(r, carry):
            for k in range(K8):
                mbuf[r, pl.ds(k * 16, 16)] = _zeros16()
            cubuf[r, :] = _zeros16()
            return carry
        lax.fori_loop(0, C, zrow, 0)
        off = 0
        while off < RT:
            sz = min(C, RT - off)
            if fused:
                pltpu.sync_copy(mbuf.at[pl.ds(0, sz)],
                                accM.at[pl.ds(s * RT + off, sz)])
            pltpu.sync_copy(cubuf.at[pl.ds(0, sz)],
                            accC.at[pl.ds(s * RT + off, sz)])
            off += sz
        plsc.subcore_barrier()

        def chunk_body(it, carry):
            j = s + it * NS

            @pl.when(j < n_chunks)
            def _():
                e0 = c * EC + j * C
                pltpu.sync_copy(src.at[pl.ds(e0, C)], src_v)
                pltpu.sync_copy(dst.at[pl.ds(e0, C)], dst_v)
                pltpu.sync_copy(P.at[pl.ds(e0, C)], pbuf)
                pltpu.async_copy(A.at[dst_v], pbuf, sem, add=True).wait()
                pltpu.async_copy(B.at[src_v], pbuf, sem, add=True).wait()
                pltpu.async_copy(Xp.at[dst_v], xibuf, sem).wait()
                pltpu.async_copy(Xp.at[src_v], xjbuf, sem).wait()

                def edge(e, carry2):
                    rel = xibuf[e, :] - xjbuf[e, :]
                    d2 = _hsum_all(rel * rel)
                    accv = _zeros16()
                    for k in range(K8):
                        sl = pl.ds(k * 16, 16)
                        t = pbuf[e, sl] + d2 * cv[sl]
                        mk = t / (1.0 + jnp.exp(-t))
                        mbuf[e, sl] = mk
                        accv = accv + mk * cv[pl.ds(od + k * 16, 16)]
                    tv = _hsum_all(accv) + cv[pl.ds(2 * od, 16)]
                    co = tv / (1.0 + jnp.exp(-tv))
                    cubuf[e, :] = rel * co
                    return carry2
                lax.fori_loop(0, C, edge, 0)

                if fused:
                    pltpu.sync_copy(mbuf, accM.at[dst_v], add=True)
                else:
                    pltpu.sync_copy(mbuf, out1.at[pl.ds(e0, C)])
                pltpu.sync_copy(cubuf, accC.at[dst_v], add=True)
            return carry
        lax.fori_loop(0, iters, chunk_body, 0)
        plsc.subcore_barrier()

        if fused:
            pltpu.sync_copy(accM.at[pl.ds(s * RT, RT)],
                            out1.at[c, pl.ds(s * RT, RT)])
        pltpu.sync_copy(accC.at[pl.ds(s * RT, RT)],
                        out2.at[c, pl.ds(s * RT, RT)])

    return kern


@functools.cache
def _make_scatter_l0(od):
    C = 128
    half = od // NC
    n_chunks = E // C
    iters = -(-n_chunks // NS)

    mesh = plsc.VectorSubcoreMesh(core_axis_name="c", subcore_axis_name="s")

    @functools.partial(
        pl.kernel,
        out_type=jax.ShapeDtypeStruct((NP, od), jnp.float32),
        mesh=mesh,
        scratch_types=[
            pltpu.VMEM((C,), jnp.int32),
            pltpu.VMEM((C, half), jnp.float32),
            pltpu.VMEM_SHARED((NP, half), jnp.float32),
        ],
        compiler_params=pltpu.CompilerParams(needs_layout_passes=False,
                                             use_tc_tiling_on_sc=False))
    def kern(m, dst, out, dst_v, mbuf, accM):
        c = lax.axis_index("c")
        s = lax.axis_index("s")

        def zrow(r, carry):
            for k in range(half // 16):
                mbuf[r, pl.ds(k * 16, 16)] = _zeros16()
            return carry
        lax.fori_loop(0, min(C, RT), zrow, 0)
        off = 0
        while off < RT:
            sz = min(C, RT - off)
            pltpu.sync_copy(mbuf.at[pl.ds(0, sz)],
                            accM.at[pl.ds(s * RT + off, sz)])
            off += sz
        plsc.subcore_barrier()

        def chunk_body(it, carry):
            j = s + it * NS

            @pl.when(j < n_chunks)
            def _():
                e0 = j * C
                pltpu.sync_copy(dst.at[pl.ds(e0, C)], dst_v)
                pltpu.sync_copy(m.at[pl.ds(e0, C), pl.ds(c * half, half)],
                                mbuf)
                pltpu.sync_copy(mbuf, accM.at[dst_v], add=True)
            return carry
        lax.fori_loop(0, iters, chunk_body, 0)
        plsc.subcore_barrier()

        pltpu.sync_copy(accM.at[pl.ds(s * RT, RT)],
                        out.at[pl.ds(s * RT, RT), pl.ds(c * half, half)])

    return kern


def kernel(h, pos, edge_attr, params, edge_index):
    src = edge_index[0]
    dst = edge_index[1]
    xp = jnp.pad(pos, ((0, 0), (0, 13)))

    Was = []
    for (W_e, *_rest) in params:
        hd = (W_e.shape[0] - 301) // 2
        Was.append(W_e[2 * hd + 1:])
    Ps = _edge_proj(edge_attr, Was)

    h_cur = h
    x_cur = xp
    for i, p in enumerate(params):
        W_e, b_e, W_c, b_c, W_n, b_n, cs, gamma, beta = p
        hd = int(h_cur.shape[1])
        od = int(W_e.shape[1])
        A, B = _node_ab(h_cur, W_e[:hd], W_e[hd:2 * hd], b_e)
        consts = jnp.concatenate(
            [W_e[2 * hd], W_c[:, 0], jnp.full((16,), b_c, jnp.float32)])

        fused = od * NP * 4 <= 6 * 1024 * 1024
        C = 64
        if fused:
            aggm, aggc = _make_edge_sc(od, C, True)(
                A, B, x_cur, Ps[i], src, dst, consts)
        else:
            m, aggc = _make_edge_sc(od, C, False)(
                A, B, x_cur, Ps[i], src, dst, consts)
            aggm = _make_scatter_l0(od)(m, dst)
            aggm = aggm.reshape(1, NP, od)

        u, x_cur = _node_matmul(
            h_cur, aggm, aggc, x_cur, W_n[:hd], W_n[hd:], b_n, cs)
        h_cur = _bn_act(u, gamma, beta, act=(i < 3))
    return h_cur


# speedup vs baseline: 1.1371x; 1.1371x over previous
import functools

import jax
import jax.numpy as jnp
from jax import lax
from jax.experimental import pallas as pl
from jax.experimental.pallas import tpu as pltpu
from jax.experimental.pallas import tpu_sc as plsc

N = 10000
E = 160000
NC = 2
NS = 16
RT = 632
NP = NS * RT


def _edge_proj(edge_attr, Ws):
    BE = 2000
    ods = [int(w.shape[1]) for w in Ws]

    def body(ea_ref, *refs):
        w_refs = refs[:len(ods)]
        o_refs = refs[len(ods):]
        a = ea_ref[...]
        for w, o in zip(w_refs, o_refs):
            o[...] = jnp.dot(a, w[...], preferred_element_type=jnp.float32)

    return pl.pallas_call(
        body,
        grid=(E // BE,),
        in_specs=[pl.BlockSpec((BE, 300), lambda i: (i, 0))]
        + [pl.BlockSpec((300, od), lambda i: (0, 0)) for od in ods],
        out_specs=[pl.BlockSpec((BE, od), lambda i: (i, 0)) for od in ods],
        out_shape=[jax.ShapeDtypeStruct((E, od), jnp.float32) for od in ods],
    )(edge_attr, *Ws)


def _node_ab(h, Wi, Wj, be):
    hd = int(h.shape[1])
    od = int(Wi.shape[1])
    BN_ = 2000

    def body(h_ref, wi_ref, wj_ref, be_ref, a_ref, b_ref):
        hh = h_ref[...]
        a_ref[...] = jnp.dot(hh, wi_ref[...], preferred_element_type=jnp.float32) + be_ref[...]
        b_ref[...] = jnp.dot(hh, wj_ref[...], preferred_element_type=jnp.float32)

    return pl.pallas_call(
        body,
        grid=(N // BN_,),
        in_specs=[
            pl.BlockSpec((BN_, hd), lambda i: (i, 0)),
            pl.BlockSpec((hd, od), lambda i: (0, 0)),
            pl.BlockSpec((hd, od), lambda i: (0, 0)),
            pl.BlockSpec((1, od), lambda i: (0, 0)),
        ],
        out_specs=[
            pl.BlockSpec((BN_, od), lambda i: (i, 0)),
            pl.BlockSpec((BN_, od), lambda i: (i, 0)),
        ],
        out_shape=[
            jax.ShapeDtypeStruct((N, od), jnp.float32),
            jax.ShapeDtypeStruct((N, od), jnp.float32),
        ],
    )(h, Wi, Wj, be.reshape(1, od))


def _node_matmul(h, aggm, aggc, xp, Wh, Wa, bn, cs):
    hd = int(h.shape[1])
    od = int(Wa.shape[0])
    nm = int(aggm.shape[0])
    BR = 2000

    def body(h_ref, am_ref, ac_ref, x_ref, wh_ref, wa_ref, bn_ref, cs_ref,
             u_ref, xo_ref):
        aggM = am_ref[0]
        for k in range(1, nm):
            aggM = aggM + am_ref[k]
        aggC = ac_ref[0] + ac_ref[1]
        xo_ref[...] = x_ref[...] + cs_ref[0, 0] * aggC
        z = (jnp.dot(h_ref[...], wh_ref[...], preferred_element_type=jnp.float32)
             + jnp.dot(aggM, wa_ref[...], preferred_element_type=jnp.float32)
             + bn_ref[...])
        u_ref[...] = z * jax.nn.sigmoid(z)

    return pl.pallas_call(
        body,
        grid=(N // BR,),
        in_specs=[
            pl.BlockSpec((BR, hd), lambda i: (i, 0)),
            pl.BlockSpec((nm, BR, od), lambda i: (0, i, 0)),
            pl.BlockSpec((NC, BR, 16), lambda i: (0, i, 0)),
            pl.BlockSpec((BR, 16), lambda i: (i, 0)),
            pl.BlockSpec((hd, od), lambda i: (0, 0)),
            pl.BlockSpec((od, od), lambda i: (0, 0)),
            pl.BlockSpec((1, od), lambda i: (0, 0)),
            pl.BlockSpec((1, 1), lambda i: (0, 0)),
        ],
        out_specs=[
            pl.BlockSpec((BR, od), lambda i: (i, 0)),
            pl.BlockSpec((BR, 16), lambda i: (i, 0)),
        ],
        out_shape=[
            jax.ShapeDtypeStruct((N, od), jnp.float32),
            jax.ShapeDtypeStruct((N, 16), jnp.float32),
        ],
    )(h, aggm, aggc, xp, Wh, Wa, bn.reshape(1, od), cs.reshape(1, 1))


def _bn_act(u, gamma, beta, act):
    od = int(u.shape[1])
    BOD = min(od, 128)

    def body(u_ref, gm_ref, bt_ref, ho_ref):
        uu = u_ref[...]
        mu = jnp.mean(uu, axis=0, keepdims=True)
        d = uu - mu
        var = jnp.mean(d * d, axis=0, keepdims=True)
        v = d * lax.rsqrt(var + 1e-5) * gm_ref[...] + bt_ref[...]
        if act:
            v = v * jax.nn.sigmoid(v)
        ho_ref[...] = v

    return pl.pallas_call(
        body,
        grid=(od // BOD,),
        in_specs=[
            pl.BlockSpec((N, BOD), lambda i: (0, i)),
            pl.BlockSpec((1, BOD), lambda i: (0, i)),
            pl.BlockSpec((1, BOD), lambda i: (0, i)),
        ],
        out_specs=pl.BlockSpec((N, BOD), lambda i: (0, i)),
        out_shape=jax.ShapeDtypeStruct((N, od), jnp.float32),
    )(u, gamma.reshape(1, od), beta.reshape(1, od))


def _zeros16():
    return jnp.zeros((16,), jnp.float32)


def _round_bf16(v):
    b = lax.bitcast_convert_type(v, jnp.int32)
    r = b + 0x7FFF + lax.bitwise_and(lax.shift_right_logical(b, 16), 1)
    r = lax.bitwise_and(r, -65536)
    return lax.bitcast_convert_type(r, jnp.float32)


def _hsum_all(v):
    lanes = lax.iota(jnp.int32, 16)
    for sh in (8, 4, 2, 1):
        idx = lax.bitwise_xor(lanes, sh)
        v = v + v.at[idx].get(mode="promise_in_bounds")
    return v


@functools.cache
def _make_edge_sc(od, C, fused):
    EC = E // NC
    n_chunks = EC // C
    iters = -(-n_chunks // NS)
    K8 = od // 16

    mesh = plsc.VectorSubcoreMesh(core_axis_name="c", subcore_axis_name="s")

    if fused:
        out_type = [jax.ShapeDtypeStruct((NC, NP, od), jnp.float32),
                    jax.ShapeDtypeStruct((NC, NP, 16), jnp.float32)]
    else:
        out_type = [jax.ShapeDtypeStruct((E, od), jnp.float32),
                    jax.ShapeDtypeStruct((NC, NP, 16), jnp.float32)]

    scratch = [
        pltpu.VMEM((C,), jnp.int32),
        pltpu.VMEM((C,), jnp.int32),
        pltpu.VMEM((C, od), jnp.float32),
        pltpu.VMEM((C, 16), jnp.float32),
        pltpu.VMEM((C, 16), jnp.float32),
        pltpu.VMEM((C, 16), jnp.float32),
        pltpu.VMEM((2 * od + 16,), jnp.float32),
        pltpu.SemaphoreType.DMA,
    ]
    if fused:
        scratch.append(pltpu.VMEM_SHARED((NP, od), jnp.float32))
    scratch.append(pltpu.VMEM_SHARED((NP, 16), jnp.float32))

    @functools.partial(pl.kernel, out_type=out_type, mesh=mesh,
                       scratch_types=scratch,
                       compiler_params=pltpu.CompilerParams(
                           needs_layout_passes=False,
                           use_tc_tiling_on_sc=False))
    def kern(A, B, Xp, P, src, dst, consts, out1, out2,
             src_v, dst_v, pbuf, xibuf, xjbuf, cubuf,
             cv, sem, *shared):
        if fused:
            accM, accC = shared
        else:
            (accC,) = shared
        c = lax.axis_index("c")
        s = lax.axis_index("s")

        pltpu.sync_copy(consts, cv)

        def zrow(r, carry):
            for k in range(K8):
                pbuf[r, pl.ds(k * 16, 16)] = _zeros16()
            cubuf[r, :] = _zeros16()
            return carry
        lax.fori_loop(0, C, zrow, 0)
        off = 0
        while off < RT:
            sz = min(C, RT - off)
            if fused:
                pltpu.sync_copy(pbuf.at[pl.ds(0, sz)],
                                accM.at[pl.ds(s * RT + off, sz)])
            pltpu.sync_copy(cubuf.at[pl.ds(0, sz)],
                            accC.at[pl.ds(s * RT + off, sz)])
            off += sz
        plsc.subcore_barrier()

        def chunk_body(it, carry):
            j = s + it * NS

            @pl.when(j < n_chunks)
            def _():
                e0 = c * EC + j * C
                d1 = pltpu.async_copy(src.at[pl.ds(e0, C)], src_v, sem)
                d2_ = pltpu.async_copy(dst.at[pl.ds(e0, C)], dst_v, sem)
                dp = pltpu.async_copy(P.at[pl.ds(e0, C)], pbuf, sem)
                d1.wait()
                d2_.wait()
                dp.wait()
                g1 = pltpu.async_copy(A.at[dst_v], pbuf, sem, add=True)
                g2 = pltpu.async_copy(B.at[src_v], pbuf, sem, add=True)
                g3 = pltpu.async_copy(Xp.at[dst_v], xibuf, sem)
                g4 = pltpu.async_copy(Xp.at[src_v], xjbuf, sem)
                g1.wait()
                g2.wait()
                g3.wait()
                g4.wait()

                def edge(e, carry2):
                    rel = xibuf[e, :] - xjbuf[e, :]
                    d2 = _hsum_all(rel * rel)
                    accv = _zeros16()
                    for k in range(K8):
                        sl = pl.ds(k * 16, 16)
                        t = pbuf[e, sl] + d2 * cv[sl]
                        mk = t / (1.0 + jnp.exp(-t))
                        pbuf[e, sl] = mk
                        accv = accv + mk * cv[pl.ds(od + k * 16, 16)]
                    tv = _hsum_all(accv) + cv[pl.ds(2 * od, 16)]
                    co = tv / (1.0 + jnp.exp(-tv))
                    cubuf[e, :] = rel * co
                    return carry2
                lax.fori_loop(0, C, edge, 0)

                if fused:
                    pltpu.sync_copy(pbuf, accM.at[dst_v], add=True)
                else:
                    pltpu.sync_copy(pbuf, out1.at[pl.ds(e0, C)])
                pltpu.sync_copy(cubuf, accC.at[dst_v], add=True)
            return carry
        lax.fori_loop(0, iters, chunk_body, 0)
        plsc.subcore_barrier()

        if fused:
            pltpu.sync_copy(accM.at[pl.ds(s * RT, RT)],
                            out1.at[c, pl.ds(s * RT, RT)])
        pltpu.sync_copy(accC.at[pl.ds(s * RT, RT)],
                        out2.at[c, pl.ds(s * RT, RT)])

    return kern


@functools.cache
def _make_scatter_l0(od):
    C = 128
    half = od // NC
    n_chunks = E // C
    iters = -(-n_chunks // NS)

    mesh = plsc.VectorSubcoreMesh(core_axis_name="c", subcore_axis_name="s")

    @functools.partial(
        pl.kernel,
        out_type=jax.ShapeDtypeStruct((NP, od), jnp.float32),
        mesh=mesh,
        scratch_types=[
            pltpu.VMEM((C,), jnp.int32),
            pltpu.VMEM((C, half), jnp.float32),
            pltpu.VMEM_SHARED((NP, half), jnp.float32),
        ],
        compiler_params=pltpu.CompilerParams(needs_layout_passes=False,
                                             use_tc_tiling_on_sc=False))
    def kern(m, dst, out, dst_v, mbuf, accM):
        c = lax.axis_index("c")
        s = lax.axis_index("s")

        def zrow(r, carry):
            for k in range(half // 16):
                mbuf[r, pl.ds(k * 16, 16)] = _zeros16()
            return carry
        lax.fori_loop(0, min(C, RT), zrow, 0)
        off = 0
        while off < RT:
            sz = min(C, RT - off)
            pltpu.sync_copy(mbuf.at[pl.ds(0, sz)],
                            accM.at[pl.ds(s * RT + off, sz)])
            off += sz
        plsc.subcore_barrier()

        def chunk_body(it, carry):
            j = s + it * NS

            @pl.when(j < n_chunks)
            def _():
                e0 = j * C
                pltpu.sync_copy(dst.at[pl.ds(e0, C)], dst_v)
                pltpu.sync_copy(m.at[pl.ds(e0, C), pl.ds(c * half, half)],
                                mbuf)
                pltpu.sync_copy(mbuf, accM.at[dst_v], add=True)
            return carry
        lax.fori_loop(0, iters, chunk_body, 0)
        plsc.subcore_barrier()

        pltpu.sync_copy(accM.at[pl.ds(s * RT, RT)],
                        out.at[pl.ds(s * RT, RT), pl.ds(c * half, half)])

    return kern


def kernel(h, pos, edge_attr, params, edge_index):
    src = edge_index[0]
    dst = edge_index[1]
    xp = jnp.pad(pos, ((0, 0), (0, 13)))

    Was = []
    for (W_e, *_rest) in params:
        hd = (W_e.shape[0] - 301) // 2
        Was.append(W_e[2 * hd + 1:])
    Ps = _edge_proj(edge_attr, Was)

    h_cur = h
    x_cur = xp
    for i, p in enumerate(params):
        W_e, b_e, W_c, b_c, W_n, b_n, cs, gamma, beta = p
        hd = int(h_cur.shape[1])
        od = int(W_e.shape[1])
        A, B = _node_ab(h_cur, W_e[:hd], W_e[hd:2 * hd], b_e)
        consts = jnp.concatenate(
            [W_e[2 * hd], W_c[:, 0], jnp.full((16,), b_c, jnp.float32)])

        fused = od * NP * 4 <= 6 * 1024 * 1024
        C = 128
        if fused:
            aggm, aggc = _make_edge_sc(od, C, True)(
                A, B, x_cur, Ps[i], src, dst, consts)
        else:
            m, aggc = _make_edge_sc(od, C, False)(
                A, B, x_cur, Ps[i], src, dst, consts)
            aggm = _make_scatter_l0(od)(m, dst)
            aggm = aggm.reshape(1, NP, od)

        u, x_cur = _node_matmul(
            h_cur, aggm, aggc, x_cur, W_n[:hd], W_n[hd:], b_n, cs)
        h_cur = _bn_act(u, gamma, beta, act=(i < 3))
    return h_cur


# speedup vs baseline: 1.1686x; 1.0277x over previous
import functools

import jax
import jax.numpy as jnp
from jax import lax
from jax.experimental import pallas as pl
from jax.experimental.pallas import tpu as pltpu
from jax.experimental.pallas import tpu_sc as plsc

N = 10000
E = 160000
NC = 2
NS = 16
RT = 632
NP = NS * RT


def _edge_proj(edge_attr, Ws):
    BE = 2000
    ods = [int(w.shape[1]) for w in Ws]

    def body(ea_ref, *refs):
        w_refs = refs[:len(ods)]
        o_refs = refs[len(ods):]
        a = ea_ref[...]
        for w, o in zip(w_refs, o_refs):
            o[...] = jnp.dot(a, w[...], preferred_element_type=jnp.float32)

    return pl.pallas_call(
        body,
        grid=(E // BE,),
        in_specs=[pl.BlockSpec((BE, 300), lambda i: (i, 0))]
        + [pl.BlockSpec((300, od), lambda i: (0, 0)) for od in ods],
        out_specs=[pl.BlockSpec((BE, od), lambda i: (i, 0)) for od in ods],
        out_shape=[jax.ShapeDtypeStruct((E, od), jnp.float32) for od in ods],
    )(edge_attr, *Ws)


def _node_ab(h, Wi, Wj, be):
    hd = int(h.shape[1])
    od = int(Wi.shape[1])
    BN_ = 2000

    def body(h_ref, wi_ref, wj_ref, be_ref, a_ref, b_ref):
        hh = h_ref[...]
        a_ref[...] = jnp.dot(hh, wi_ref[...], preferred_element_type=jnp.float32) + be_ref[...]
        b_ref[...] = jnp.dot(hh, wj_ref[...], preferred_element_type=jnp.float32)

    return pl.pallas_call(
        body,
        grid=(N // BN_,),
        in_specs=[
            pl.BlockSpec((BN_, hd), lambda i: (i, 0)),
            pl.BlockSpec((hd, od), lambda i: (0, 0)),
            pl.BlockSpec((hd, od), lambda i: (0, 0)),
            pl.BlockSpec((1, od), lambda i: (0, 0)),
        ],
        out_specs=[
            pl.BlockSpec((BN_, od), lambda i: (i, 0)),
            pl.BlockSpec((BN_, od), lambda i: (i, 0)),
        ],
        out_shape=[
            jax.ShapeDtypeStruct((N, od), jnp.float32),
            jax.ShapeDtypeStruct((N, od), jnp.float32),
        ],
    )(h, Wi, Wj, be.reshape(1, od))


def _node_matmul(h, aggm, aggc, xp, Wh, Wa, bn, cs):
    hd = int(h.shape[1])
    od = int(Wa.shape[0])
    nm = int(aggm.shape[0])
    BR = 2000

    def body(h_ref, am_ref, ac_ref, x_ref, wh_ref, wa_ref, bn_ref, cs_ref,
             u_ref, xo_ref):
        aggM = am_ref[0]
        for k in range(1, nm):
            aggM = aggM + am_ref[k]
        aggC = ac_ref[0] + ac_ref[1]
        xo_ref[...] = x_ref[...] + cs_ref[0, 0] * aggC
        z = (jnp.dot(h_ref[...], wh_ref[...], preferred_element_type=jnp.float32)
             + jnp.dot(aggM, wa_ref[...], preferred_element_type=jnp.float32)
             + bn_ref[...])
        u_ref[...] = z * jax.nn.sigmoid(z)

    return pl.pallas_call(
        body,
        grid=(N // BR,),
        in_specs=[
            pl.BlockSpec((BR, hd), lambda i: (i, 0)),
            pl.BlockSpec((nm, BR, od), lambda i: (0, i, 0)),
            pl.BlockSpec((NC, BR, 16), lambda i: (0, i, 0)),
            pl.BlockSpec((BR, 16), lambda i: (i, 0)),
            pl.BlockSpec((hd, od), lambda i: (0, 0)),
            pl.BlockSpec((od, od), lambda i: (0, 0)),
            pl.BlockSpec((1, od), lambda i: (0, 0)),
            pl.BlockSpec((1, 1), lambda i: (0, 0)),
        ],
        out_specs=[
            pl.BlockSpec((BR, od), lambda i: (i, 0)),
            pl.BlockSpec((BR, 16), lambda i: (i, 0)),
        ],
        out_shape=[
            jax.ShapeDtypeStruct((N, od), jnp.float32),
            jax.ShapeDtypeStruct((N, 16), jnp.float32),
        ],
    )(h, aggm, aggc, xp, Wh, Wa, bn.reshape(1, od), cs.reshape(1, 1))


def _bn_act(u, gamma, beta, act):
    od = int(u.shape[1])
    BOD = min(od, 128)

    def body(u_ref, gm_ref, bt_ref, ho_ref):
        uu = u_ref[...]
        mu = jnp.mean(uu, axis=0, keepdims=True)
        d = uu - mu
        var = jnp.mean(d * d, axis=0, keepdims=True)
        v = d * lax.rsqrt(var + 1e-5) * gm_ref[...] + bt_ref[...]
        if act:
            v = v * jax.nn.sigmoid(v)
        ho_ref[...] = v

    return pl.pallas_call(
        body,
        grid=(od // BOD,),
        in_specs=[
            pl.BlockSpec((N, BOD), lambda i: (0, i)),
            pl.BlockSpec((1, BOD), lambda i: (0, i)),
            pl.BlockSpec((1, BOD), lambda i: (0, i)),
        ],
        out_specs=pl.BlockSpec((N, BOD), lambda i: (0, i)),
        out_shape=jax.ShapeDtypeStruct((N, od), jnp.float32),
    )(u, gamma.reshape(1, od), beta.reshape(1, od))


def _zeros16():
    return jnp.zeros((16,), jnp.float32)


def _round_bf16(v):
    b = lax.bitcast_convert_type(v, jnp.int32)
    r = b + 0x7FFF + lax.bitwise_and(lax.shift_right_logical(b, 16), 1)
    r = lax.bitwise_and(r, -65536)
    return lax.bitcast_convert_type(r, jnp.float32)


def _hsum_all(v):
    lanes = lax.iota(jnp.int32, 16)
    for sh in (8, 4, 2, 1):
        idx = lax.bitwise_xor(lanes, sh)
        v = v + v.at[idx].get(mode="promise_in_bounds")
    return v


@functools.cache
def _make_edge_sc(od, C, fused):
    EC = E // NC
    n_chunks = EC // C
    iters = -(-n_chunks // NS)
    K8 = od // 16

    mesh = plsc.VectorSubcoreMesh(core_axis_name="c", subcore_axis_name="s")

    if fused:
        out_type = [jax.ShapeDtypeStruct((NC, NP, od), jnp.float32),
                    jax.ShapeDtypeStruct((NC, NP, 16), jnp.float32)]
    else:
        out_type = [jax.ShapeDtypeStruct((E, od), jnp.float32),
                    jax.ShapeDtypeStruct((NC, NP, 16), jnp.float32)]

    scratch = [
        pltpu.VMEM((C,), jnp.int32),
        pltpu.VMEM((C,), jnp.int32),
        pltpu.VMEM((C, od), jnp.float32),
        pltpu.VMEM((C, 16), jnp.float32),
        pltpu.VMEM((C, 16), jnp.float32),
        pltpu.VMEM((C, 16), jnp.float32),
        pltpu.VMEM((2 * od + 16,), jnp.float32),
        pltpu.SemaphoreType.DMA,
    ]
    if fused:
        scratch.append(pltpu.VMEM_SHARED((NP, od), jnp.float32))
    scratch.append(pltpu.VMEM_SHARED((NP, 16), jnp.float32))

    @functools.partial(pl.kernel, out_type=out_type, mesh=mesh,
                       scratch_types=scratch,
                       compiler_params=pltpu.CompilerParams(
                           needs_layout_passes=False,
                           use_tc_tiling_on_sc=False))
    def kern(A, B, Xp, P, src, dst, consts, out1, out2,
             src_v, dst_v, pbuf, xibuf, xjbuf, cubuf,
             cv, sem, *shared):
        if fused:
            accM, accC = shared
        else:
            (accC,) = shared
        c = lax.axis_index("c")
        s = lax.axis_index("s")

        pltpu.sync_copy(consts, cv)

        def zrow(r, carry):
            for k in range(K8):
                pbuf[r, pl.ds(k * 16, 16)] = _zeros16()
            cubuf[r, :] = _zeros16()
            return carry
        lax.fori_loop(0, C, zrow, 0)
        off = 0
        while off < RT:
            sz = min(C, RT - off)
            if fused:
                pltpu.sync_copy(pbuf.at[pl.ds(0, sz)],
                                accM.at[pl.ds(s * RT + off, sz)])
            pltpu.sync_copy(cubuf.at[pl.ds(0, sz)],
                            accC.at[pl.ds(s * RT + off, sz)])
            off += sz
        plsc.subcore_barrier()

        def chunk_body(it, carry):
            j = s + it * NS

            @pl.when(j < n_chunks)
            def _():
                e0 = c * EC + j * C
                d1 = pltpu.async_copy(src.at[pl.ds(e0, C)], src_v, sem)
                d2_ = pltpu.async_copy(dst.at[pl.ds(e0, C)], dst_v, sem)
                dp = pltpu.async_copy(P.at[pl.ds(e0, C)], pbuf, sem)
                d1.wait()
                d2_.wait()
                dp.wait()
                g1 = pltpu.async_copy(A.at[dst_v], pbuf, sem, add=True)
                g2 = pltpu.async_copy(B.at[src_v], pbuf, sem, add=True)
                g3 = pltpu.async_copy(Xp.at[dst_v], xibuf, sem)
                g4 = pltpu.async_copy(Xp.at[src_v], xjbuf, sem)
                g1.wait()
                g2.wait()
                g3.wait()
                g4.wait()

                def edge4(eb, carry2):
                    for u in range(4):
                        e = eb * 4 + u
                        rel = xibuf[e, :] - xjbuf[e, :]
                        d2 = _hsum_all(rel * rel)
                        accv = _zeros16()
                        for k in range(K8):
                            sl = pl.ds(k * 16, 16)
                            t = pbuf[e, sl] + d2 * cv[sl]
                            mk = t / (1.0 + jnp.exp(-t))
                            pbuf[e, sl] = mk
                            accv = accv + mk * cv[pl.ds(od + k * 16, 16)]
                        tv = _hsum_all(accv) + cv[pl.ds(2 * od, 16)]
                        co = tv / (1.0 + jnp.exp(-tv))
                        cubuf[e, :] = rel * co
                    return carry2
                lax.fori_loop(0, C // 4, edge4, 0)

                if fused:
                    pltpu.sync_copy(pbuf, accM.at[dst_v], add=True)
                else:
                    pltpu.sync_copy(pbuf, out1.at[pl.ds(e0, C)])
                pltpu.sync_copy(cubuf, accC.at[dst_v], add=True)
            return carry
        lax.fori_loop(0, iters, chunk_body, 0)
        plsc.subcore_barrier()

        if fused:
            pltpu.sync_copy(accM.at[pl.ds(s * RT, RT)],
                            out1.at[c, pl.ds(s * RT, RT)])
        pltpu.sync_copy(accC.at[pl.ds(s * RT, RT)],
                        out2.at[c, pl.ds(s * RT, RT)])

    return kern


@functools.cache
def _make_scatter_l0(od):
    C = 128
    half = od // NC
    n_chunks = E // C
    iters = -(-n_chunks // NS)

    mesh = plsc.VectorSubcoreMesh(core_axis_name="c", subcore_axis_name="s")

    @functools.partial(
        pl.kernel,
        out_type=jax.ShapeDtypeStruct((NP, od), jnp.float32),
        mesh=mesh,
        scratch_types=[
            pltpu.VMEM((C,), jnp.int32),
            pltpu.VMEM((C, half), jnp.float32),
            pltpu.VMEM_SHARED((NP, half), jnp.float32),
        ],
        compiler_params=pltpu.CompilerParams(needs_layout_passes=False,
                                             use_tc_tiling_on_sc=False))
    def kern(m, dst, out, dst_v, mbuf, accM):
        c = lax.axis_index("c")
        s = lax.axis_index("s")

        def zrow(r, carry):
            for k in range(half // 16):
                mbuf[r, pl.ds(k * 16, 16)] = _zeros16()
            return carry
        lax.fori_loop(0, min(C, RT), zrow, 0)
        off = 0
        while off < RT:
            sz = min(C, RT - off)
            pltpu.sync_copy(mbuf.at[pl.ds(0, sz)],
                            accM.at[pl.ds(s * RT + off, sz)])
            off += sz
        plsc.subcore_barrier()

        def chunk_body(it, carry):
            j = s + it * NS

            @pl.when(j < n_chunks)
            def _():
                e0 = j * C
                pltpu.sync_copy(dst.at[pl.ds(e0, C)], dst_v)
                pltpu.sync_copy(m.at[pl.ds(e0, C), pl.ds(c * half, half)],
                                mbuf)
                pltpu.sync_copy(mbuf, accM.at[dst_v], add=True)
            return carry
        lax.fori_loop(0, iters, chunk_body, 0)
        plsc.subcore_barrier()

        pltpu.sync_copy(accM.at[pl.ds(s * RT, RT)],
                        out.at[pl.ds(s * RT, RT), pl.ds(c * half, half)])

    return kern


def kernel(h, pos, edge_attr, params, edge_index):
    src = edge_index[0]
    dst = edge_index[1]
    xp = jnp.pad(pos, ((0, 0), (0, 13)))

    Was = []
    for (W_e, *_rest) in params:
        hd = (W_e.shape[0] - 301) // 2
        Was.append(W_e[2 * hd + 1:])
    Ps = _edge_proj(edge_attr, Was)

    h_cur = h
    x_cur = xp
    for i, p in enumerate(params):
        W_e, b_e, W_c, b_c, W_n, b_n, cs, gamma, beta = p
        hd = int(h_cur.shape[1])
        od = int(W_e.shape[1])
        A, B = _node_ab(h_cur, W_e[:hd], W_e[hd:2 * hd], b_e)
        consts = jnp.concatenate(
            [W_e[2 * hd], W_c[:, 0], jnp.full((16,), b_c, jnp.float32)])

        fused = od * NP * 4 <= 6 * 1024 * 1024
        C = 128
        if fused:
            aggm, aggc = _make_edge_sc(od, C, True)(
                A, B, x_cur, Ps[i], src, dst, consts)
        else:
            m, aggc = _make_edge_sc(od, C, False)(
                A, B, x_cur, Ps[i], src, dst, consts)
            aggm = _make_scatter_l0(od)(m, dst)
            aggm = aggm.reshape(1, NP, od)

        u, x_cur = _node_matmul(
            h_cur, aggm, aggc, x_cur, W_n[:hd], W_n[hd:], b_n, cs)
        h_cur = _bn_act(u, gamma, beta, act=(i < 3))
    return h_cur


# speedup vs baseline: 1.1686x; 1.0000x over previous
import functools

import jax
import jax.numpy as jnp
from jax import lax
from jax.experimental import pallas as pl
from jax.experimental.pallas import tpu as pltpu
from jax.experimental.pallas import tpu_sc as plsc

N = 10000
E = 160000
NC = 2
NS = 16
RT = 632
NP = NS * RT


def _edge_proj(edge_attr, Ws):
    BE = 2000
    ods = [int(w.shape[1]) for w in Ws]

    def body(ea_ref, *refs):
        w_refs = refs[:len(ods)]
        o_refs = refs[len(ods):]
        a = ea_ref[...]
        for w, o in zip(w_refs, o_refs):
            o[...] = jnp.dot(a, w[...], preferred_element_type=jnp.float32)

    return pl.pallas_call(
        body,
        grid=(E // BE,),
        in_specs=[pl.BlockSpec((BE, 300), lambda i: (i, 0))]
        + [pl.BlockSpec((300, od), lambda i: (0, 0)) for od in ods],
        out_specs=[pl.BlockSpec((BE, od), lambda i: (i, 0)) for od in ods],
        out_shape=[jax.ShapeDtypeStruct((E, od), jnp.float32) for od in ods],
    )(edge_attr, *Ws)


def _node_ab(h, Wi, Wj, be):
    hd = int(h.shape[1])
    od = int(Wi.shape[1])
    BN_ = 2000

    def body(h_ref, wi_ref, wj_ref, be_ref, a_ref, b_ref):
        hh = h_ref[...]
        a_ref[...] = jnp.dot(hh, wi_ref[...], preferred_element_type=jnp.float32) + be_ref[...]
        b_ref[...] = jnp.dot(hh, wj_ref[...], preferred_element_type=jnp.float32)

    return pl.pallas_call(
        body,
        grid=(N // BN_,),
        in_specs=[
            pl.BlockSpec((BN_, hd), lambda i: (i, 0)),
            pl.BlockSpec((hd, od), lambda i: (0, 0)),
            pl.BlockSpec((hd, od), lambda i: (0, 0)),
            pl.BlockSpec((1, od), lambda i: (0, 0)),
        ],
        out_specs=[
            pl.BlockSpec((BN_, od), lambda i: (i, 0)),
            pl.BlockSpec((BN_, od), lambda i: (i, 0)),
        ],
        out_shape=[
            jax.ShapeDtypeStruct((N, od), jnp.float32),
            jax.ShapeDtypeStruct((N, od), jnp.float32),
        ],
    )(h, Wi, Wj, be.reshape(1, od))


def _node_matmul(h, aggm, aggc, xp, Wh, Wa, bn, cs):
    hd = int(h.shape[1])
    od = int(Wa.shape[0])
    nm = int(aggm.shape[0])
    BR = 2000

    def body(h_ref, am_ref, ac_ref, x_ref, wh_ref, wa_ref, bn_ref, cs_ref,
             u_ref, xo_ref):
        aggM = am_ref[0]
        for k in range(1, nm):
            aggM = aggM + am_ref[k]
        aggC = ac_ref[0] + ac_ref[1]
        xo_ref[...] = x_ref[...] + cs_ref[0, 0] * aggC
        z = (jnp.dot(h_ref[...], wh_ref[...], preferred_element_type=jnp.float32)
             + jnp.dot(aggM, wa_ref[...], preferred_element_type=jnp.float32)
             + bn_ref[...])
        u_ref[...] = z * jax.nn.sigmoid(z)

    return pl.pallas_call(
        body,
        grid=(N // BR,),
        in_specs=[
            pl.BlockSpec((BR, hd), lambda i: (i, 0)),
            pl.BlockSpec((nm, BR, od), lambda i: (0, i, 0)),
            pl.BlockSpec((NC, BR, 16), lambda i: (0, i, 0)),
            pl.BlockSpec((BR, 16), lambda i: (i, 0)),
            pl.BlockSpec((hd, od), lambda i: (0, 0)),
            pl.BlockSpec((od, od), lambda i: (0, 0)),
            pl.BlockSpec((1, od), lambda i: (0, 0)),
            pl.BlockSpec((1, 1), lambda i: (0, 0)),
        ],
        out_specs=[
            pl.BlockSpec((BR, od), lambda i: (i, 0)),
            pl.BlockSpec((BR, 16), lambda i: (i, 0)),
        ],
        out_shape=[
            jax.ShapeDtypeStruct((N, od), jnp.float32),
            jax.ShapeDtypeStruct((N, 16), jnp.float32),
        ],
    )(h, aggm, aggc, xp, Wh, Wa, bn.reshape(1, od), cs.reshape(1, 1))


def _bn_act(u, gamma, beta, act):
    od = int(u.shape[1])
    BOD = min(od, 128)

    def body(u_ref, gm_ref, bt_ref, ho_ref):
        uu = u_ref[...]
        mu = jnp.mean(uu, axis=0, keepdims=True)
        d = uu - mu
        var = jnp.mean(d * d, axis=0, keepdims=True)
        v = d * lax.rsqrt(var + 1e-5) * gm_ref[...] + bt_ref[...]
        if act:
            v = v * jax.nn.sigmoid(v)
        ho_ref[...] = v

    return pl.pallas_call(
        body,
        grid=(od // BOD,),
        in_specs=[
            pl.BlockSpec((N, BOD), lambda i: (0, i)),
            pl.BlockSpec((1, BOD), lambda i: (0, i)),
            pl.BlockSpec((1, BOD), lambda i: (0, i)),
        ],
        out_specs=pl.BlockSpec((N, BOD), lambda i: (0, i)),
        out_shape=jax.ShapeDtypeStruct((N, od), jnp.float32),
    )(u, gamma.reshape(1, od), beta.reshape(1, od))


def _zeros16():
    return jnp.zeros((16,), jnp.float32)


def _hsum_all(v):
    lanes = lax.iota(jnp.int32, 16)
    for sh in (8, 4, 2, 1):
        idx = lax.bitwise_xor(lanes, sh)
        v = v + v.at[idx].get(mode="promise_in_bounds")
    return v


@functools.cache
def _make_edge_sc(od, C, fused):
    EC = E // NC
    n_chunks = EC // C
    iters = -(-n_chunks // NS)
    K8 = od // 16

    mesh = plsc.VectorSubcoreMesh(core_axis_name="c", subcore_axis_name="s")

    if fused:
        out_type = [jax.ShapeDtypeStruct((NC, NP, od), jnp.float32),
                    jax.ShapeDtypeStruct((NC, NP, 16), jnp.float32)]
    else:
        out_type = [jax.ShapeDtypeStruct((E, od), jnp.float32),
                    jax.ShapeDtypeStruct((NC, NP, 16), jnp.float32)]

    scratch = [
        pltpu.VMEM((C,), jnp.int32),
        pltpu.VMEM((C,), jnp.int32),
        pltpu.VMEM((C, od), jnp.float32),
        pltpu.VMEM((C, 16), jnp.float32),
        pltpu.VMEM((C, 16), jnp.float32),
        pltpu.VMEM((C, 16), jnp.float32),
        pltpu.VMEM((2 * od + 16,), jnp.float32),
        pltpu.SemaphoreType.DMA,
    ]
    if fused:
        scratch.append(pltpu.VMEM_SHARED((NP, od), jnp.float32))
    scratch.append(pltpu.VMEM_SHARED((NP, 16), jnp.float32))

    @functools.partial(pl.kernel, out_type=out_type, mesh=mesh,
                       scratch_types=scratch,
                       compiler_params=pltpu.CompilerParams(
                           needs_layout_passes=False,
                           use_tc_tiling_on_sc=False))
    def kern(A, B, Xp, P, src, dst, consts, out1, out2,
             src_v, dst_v, pbuf, xibuf, xjbuf, cubuf,
             cv, sem, *shared):
        if fused:
            accM, accC = shared
        else:
            (accC,) = shared
        c = lax.axis_index("c")
        s = lax.axis_index("s")

        pltpu.sync_copy(consts, cv)

        def zrow(r, carry):
            for k in range(K8):
                pbuf[r, pl.ds(k * 16, 16)] = _zeros16()
            cubuf[r, :] = _zeros16()
            return carry
        lax.fori_loop(0, C, zrow, 0)
        off = 0
        while off < RT:
            sz = min(C, RT - off)
            if fused:
                pltpu.sync_copy(pbuf.at[pl.ds(0, sz)],
                                accM.at[pl.ds(s * RT + off, sz)])
            pltpu.sync_copy(cubuf.at[pl.ds(0, sz)],
                            accC.at[pl.ds(s * RT + off, sz)])
            off += sz
        plsc.subcore_barrier()

        def chunk_body(it, carry):
            j = s + it * NS

            @pl.when(j < n_chunks)
            def _():
                e0 = c * EC + j * C
                d1 = pltpu.async_copy(src.at[pl.ds(e0, C)], src_v, sem)
                d2_ = pltpu.async_copy(dst.at[pl.ds(e0, C)], dst_v, sem)
                dp = pltpu.async_copy(P.at[pl.ds(e0, C)], pbuf, sem)
                d1.wait()
                d2_.wait()
                dp.wait()
                g1 = pltpu.async_copy(A.at[dst_v], pbuf, sem, add=True)
                g2 = pltpu.async_copy(B.at[src_v], pbuf, sem, add=True)
                g3 = pltpu.async_copy(Xp.at[dst_v], xibuf, sem)
                g4 = pltpu.async_copy(Xp.at[src_v], xjbuf, sem)
                g1.wait()
                g2.wait()
                g3.wait()
                g4.wait()

                def edge4(eb, carry2):
                    for u in range(4):
                        e = eb * 4 + u
                        rel = xibuf[e, :] - xjbuf[e, :]
                        d2 = _hsum_all(rel * rel)
                        accv = _zeros16()
                        for k in range(K8):
                            sl = pl.ds(k * 16, 16)
                            t = pbuf[e, sl] + d2 * cv[sl]
                            mk = t / (1.0 + jnp.exp(-t))
                            pbuf[e, sl] = mk
                            accv = accv + mk * cv[pl.ds(od + k * 16, 16)]
                        tv = _hsum_all(accv) + cv[pl.ds(2 * od, 16)]
                        co = tv / (1.0 + jnp.exp(-tv))
                        cubuf[e, :] = rel * co
                    return carry2
                lax.fori_loop(0, C // 4, edge4, 0)

                if fused:
                    pltpu.sync_copy(pbuf, accM.at[dst_v], add=True)
                else:
                    pltpu.sync_copy(pbuf, out1.at[pl.ds(e0, C)])
                pltpu.sync_copy(cubuf, accC.at[dst_v], add=True)
            return carry
        lax.fori_loop(0, iters, chunk_body, 0)
        plsc.subcore_barrier()

        if fused:
            pltpu.sync_copy(accM.at[pl.ds(s * RT, RT)],
                            out1.at[c, pl.ds(s * RT, RT)])
        pltpu.sync_copy(accC.at[pl.ds(s * RT, RT)],
                        out2.at[c, pl.ds(s * RT, RT)])

    return kern


@functools.cache
def _make_scatter_l0(od):
    C = 128
    half = od // NC
    n_chunks = E // C
    iters = -(-n_chunks // NS)

    mesh = plsc.VectorSubcoreMesh(core_axis_name="c", subcore_axis_name="s")

    @functools.partial(
        pl.kernel,
        out_type=jax.ShapeDtypeStruct((NP, od), jnp.float32),
        mesh=mesh,
        scratch_types=[
            pltpu.VMEM((C,), jnp.int32),
            pltpu.VMEM((C, half), jnp.float32),
            pltpu.VMEM_SHARED((NP, half), jnp.float32),
        ],
        compiler_params=pltpu.CompilerParams(needs_layout_passes=False,
                                             use_tc_tiling_on_sc=False))
    def kern(m, dst, out, dst_v, mbuf, accM):
        c = lax.axis_index("c")
        s = lax.axis_index("s")

        def zrow(r, carry):
            for k in range(half // 16):
                mbuf[r, pl.ds(k * 16, 16)] = _zeros16()
            return carry
        lax.fori_loop(0, min(C, RT), zrow, 0)
        off = 0
        while off < RT:
            sz = min(C, RT - off)
            pltpu.sync_copy(mbuf.at[pl.ds(0, sz)],
                            accM.at[pl.ds(s * RT + off, sz)])
            off += sz
        plsc.subcore_barrier()

        def chunk_body(it, carry):
            j = s + it * NS

            @pl.when(j < n_chunks)
            def _():
                e0 = j * C
                pltpu.sync_copy(dst.at[pl.ds(e0, C)], dst_v)
                pltpu.sync_copy(m.at[pl.ds(e0, C), pl.ds(c * half, half)],
                                mbuf)
                pltpu.sync_copy(mbuf, accM.at[dst_v], add=True)
            return carry
        lax.fori_loop(0, iters, chunk_body, 0)
        plsc.subcore_barrier()

        pltpu.sync_copy(accM.at[pl.ds(s * RT, RT)],
                        out.at[pl.ds(s * RT, RT), pl.ds(c * half, half)])

    return kern


def kernel(h, pos, edge_attr, params, edge_index):
    src = edge_index[0]
    dst = edge_index[1]
    xp = jnp.pad(pos, ((0, 0), (0, 13)))

    Was = []
    for (W_e, *_rest) in params:
        hd = (W_e.shape[0] - 301) // 2
        Was.append(W_e[2 * hd + 1:])
    Ps = _edge_proj(edge_attr, Was)

    h_cur = h
    x_cur = xp
    for i, p in enumerate(params):
        W_e, b_e, W_c, b_c, W_n, b_n, cs, gamma, beta = p
        hd = int(h_cur.shape[1])
        od = int(W_e.shape[1])
        A, B = _node_ab(h_cur, W_e[:hd], W_e[hd:2 * hd], b_e)
        consts = jnp.concatenate(
            [W_e[2 * hd], W_c[:, 0], jnp.full((16,), b_c, jnp.float32)])

        fused = od * NP * 4 <= 6 * 1024 * 1024
        C = 128
        if fused:
            aggm, aggc = _make_edge_sc(od, C, True)(
                A, B, x_cur, Ps[i], src, dst, consts)
        else:
            m, aggc = _make_edge_sc(od, C, False)(
                A, B, x_cur, Ps[i], src, dst, consts)
            aggm = _make_scatter_l0(od)(m, dst)
            aggm = aggm.reshape(1, NP, od)

        u, x_cur = _node_matmul(
            h_cur, aggm, aggc, x_cur, W_n[:hd], W_n[hd:], b_n, cs)
        h_cur = _bn_act(u, gamma, beta, act=(i < 3))
    return h_cur
